# baseline (speedup 1.0000x reference)
import dataclasses

import jax
import jax.numpy as jnp
from jax import lax
from jax.experimental import pallas as pl
from jax.experimental.pallas import tpu as pltpu
from jax.experimental.pallas import tpu_sc as plsc

N = 10000
E = 160000
D = 256
NLAYER = 5
EPS = 1e-5

NC = 2
NS = 16
NW = NC * NS
NB = NW
ACC_R = 320
OUTR = NB * ACC_R
EPW = E // NW
CAPP = 5120
CH = 128
NCH = CAPP // CH
DEG_ROWS = N + 240
DPS = DEG_ROWS // NS
BN_BLK = 1000

_mesh = plsc.VectorSubcoreMesh(core_axis_name="c", subcore_axis_name="s")

_sc_params = pltpu.CompilerParams()
if "needs_layout_passes" in pltpu.CompilerParams.__dataclass_fields__:
    _sc_params = dataclasses.replace(_sc_params, needs_layout_passes=False)


def _part_body(rowf, colf, ewf, prow, psid, pew, cnts, degp,
               rowb, colb, ewb, lrB, lsB, lwB,
               cntb, stage, idxb, zb, deg_sh):
    c = lax.axis_index("c")
    s = lax.axis_index("s")
    w = c * NS + s
    io = lax.iota(jnp.int32, 16)
    zeros16 = jnp.zeros((16,), jnp.float32)
    izeros16 = jnp.zeros((16,), jnp.int32)
    dummy_s = io & 7
    dummy_g = N + (io & 7)

    @pl.loop(0, DPS // 16)
    def _(i):
        zb[pl.ds(i * 16, 16)] = zeros16

    pltpu.sync_copy(zb, deg_sh.at[pl.ds(s * DPS, DPS)])

    pltpu.sync_copy(rowf.at[pl.ds(w * EPW, EPW)], rowb.at[pl.ds(0, EPW)])
    pltpu.sync_copy(colf.at[pl.ds(w * EPW, EPW)], colb.at[pl.ds(0, EPW)])
    pltpu.sync_copy(ewf.at[pl.ds(w * EPW, EPW)], ewb.at[pl.ds(0, EPW)])

    @pl.loop(0, CAPP // 16)
    def _(i):
        sl = pl.ds(i * 16, 16)
        lrB[sl] = izeros16
        lsB[sl] = dummy_s
        lwB[sl] = zeros16

    plsc.subcore_barrier()

    def deg_body(i, carry):
        sl = pl.ds(i * 16, 16)
        c16 = colb[sl]
        e16 = ewb[sl]
        valid = (i * 16 + io) < EPW
        idxb[0, pl.ds(0, 16)] = jnp.where(valid, c16, dummy_g)
        stage[...] = jnp.where(valid, e16, 0.0)
        pltpu.sync_copy(stage, deg_sh.at[idxb.at[0]], add=True)
        return carry

    lax.fori_loop(0, (EPW + 15) // 16, deg_body, jnp.int32(0))

    cv_lo = izeros16
    cv_hi = izeros16
    for t in range(NB):
        def body(i, off):
            sl = pl.ds(i * 16, 16)
            r16 = rowb[sl]
            c16 = colb[sl]
            e16 = ewb[sl]
            valid = (i * 16 + io) < EPW
            t16 = lax.div(c16, jnp.int32(ACC_R))
            m = valid & (t16 == t)
            plsc.store_compressed(lrB.at[pl.ds(off, 16)], r16, mask=m)
            plsc.store_compressed(lsB.at[pl.ds(off, 16)],
                                  c16 - (t * ACC_R), mask=m)
            plsc.store_compressed(lwB.at[pl.ds(off, 16)], e16, mask=m)
            return off + jnp.sum(m.astype(jnp.int32))

        off_t = lax.fori_loop(0, (EPW + 15) // 16, body, jnp.int32(0))

        pltpu.sync_copy(lrB, prow.at[t, w])
        pltpu.sync_copy(lsB, psid.at[t, w])
        pltpu.sync_copy(lwB, pew.at[t, w])

        def clr(i, carry):
            sl = pl.ds(i * 16, 16)
            lrB[sl] = izeros16
            lsB[sl] = dummy_s
            lwB[sl] = zeros16
            return carry

        lax.fori_loop(0, (off_t + 15) // 16, clr, jnp.int32(0))

        if t < NS:
            cv_lo = cv_lo + jnp.where(io == t, off_t, 0)
        else:
            cv_hi = cv_hi + jnp.where(io == (t - NS), off_t, 0)

    cntb[pl.ds(0, 16)] = cv_lo
    cntb[pl.ds(16, 16)] = cv_hi
    pltpu.sync_copy(cntb, cnts.at[w])

    plsc.subcore_barrier()
    pltpu.sync_copy(deg_sh.at[pl.ds(s * DPS, DPS)],
                    degp.at[c, pl.ds(s * DPS, DPS)])


def _partition(rowf, colf, ewf):
    k = pl.kernel(
        _part_body,
        out_type=[
            jax.ShapeDtypeStruct((NB, NW, CAPP), jnp.int32),
            jax.ShapeDtypeStruct((NB, NW, CAPP), jnp.int32),
            jax.ShapeDtypeStruct((NB, NW, CAPP), jnp.float32),
            jax.ShapeDtypeStruct((NW, NB), jnp.int32),
            jax.ShapeDtypeStruct((NC, DEG_ROWS), jnp.float32),
        ],
        mesh=_mesh,
        scratch_types=[
            pltpu.VMEM((EPW + 8,), jnp.int32),
            pltpu.VMEM((EPW + 8,), jnp.int32),
            pltpu.VMEM((EPW + 8,), jnp.float32),
            pltpu.VMEM((CAPP,), jnp.int32),
            pltpu.VMEM((CAPP,), jnp.int32),
            pltpu.VMEM((CAPP,), jnp.float32),
            pltpu.VMEM((32,), jnp.int32),
            pltpu.VMEM((16,), jnp.float32),
            pltpu.VMEM((1, 16), jnp.int32),
            pltpu.VMEM((DPS,), jnp.float32),
            pltpu.VMEM_SHARED((DEG_ROWS,), jnp.float32),
        ],
        compiler_params=_sc_params,
    )
    return k(rowf, colf, ewf)


def _agg_body(y, prow, psid, pew, cnts, out,
              gb, ridxb, sidxb, ewxb, cntv, acc):
    c = lax.axis_index("c")
    s = lax.axis_index("s")
    t = c * NS + s
    zeros16 = jnp.zeros((16,), jnp.float32)

    @pl.loop(0, ACC_R)
    def _(r):
        @pl.loop(0, D // 16)
        def _(k):
            acc[r, pl.ds(k * 16, 16)] = zeros16

    pltpu.sync_copy(cnts, cntv)

    @pl.loop(0, NW)
    def _(w):
        cvec = plsc.load_gather(cntv, [jnp.full((16,), NB, jnp.int32) * w + t])
        cnt = jnp.max(cvec)
        nch = (cnt + (CH - 1)) // CH

        def chunk(j, carry):
            pltpu.sync_copy(prow.at[t, w, j], ridxb.at[0])
            pltpu.sync_copy(psid.at[t, w, j], sidxb.at[0])
            pltpu.sync_copy(pew.at[t, w, j], ewxb.at[0])
            pltpu.sync_copy(y.at[ridxb.at[0]], gb)

            @pl.loop(0, CH)
            def _(e):
                e16 = jnp.full((16,), e, jnp.int32)
                ewv = plsc.load_gather(ewxb, [jnp.zeros((16,), jnp.int32), e16])
                rowe = jnp.max(plsc.load_gather(
                    sidxb, [jnp.zeros((16,), jnp.int32), e16]))
                for d in range(D // 16):
                    sl = pl.ds(d * 16, 16)
                    plsc.addupdate(acc.at[rowe, sl], ewv * gb[e, sl])

            return carry

        lax.fori_loop(0, nch, chunk, jnp.int32(0))

    pltpu.sync_copy(acc, out.at[pl.ds(t * ACC_R, ACC_R)])


def _aggregate(y, prow2, psid2, pew2, cnts):
    k = pl.kernel(
        _agg_body,
        out_type=jax.ShapeDtypeStruct((OUTR, D), jnp.float32),
        mesh=_mesh,
        scratch_types=[
            pltpu.VMEM((CH, D), jnp.float32),
            pltpu.VMEM((1, CH), jnp.int32),
            pltpu.VMEM((1, CH), jnp.int32),
            pltpu.VMEM((1, CH), jnp.float32),
            pltpu.VMEM((NW * NB,), jnp.int32),
            pltpu.VMEM((ACC_R, D), jnp.float32),
        ],
        compiler_params=_sc_params,
    )
    return k(y, prow2, psid2, pew2, cnts)


def _ml0_body(x_ref, degt_ref, emb_ref, w_ref, y_ref, dinv_ref):
    deg = jnp.sum(degt_ref[...], axis=1, keepdims=True) + 1.0
    dinv = jnp.where(deg > 0,
                     lax.rsqrt(jnp.maximum(deg, 1e-12)),
                     0.0)
    oh = (x_ref[...] == lax.broadcasted_iota(jnp.int32, (BN_BLK, 8), 1))
    h0 = jnp.dot(oh.astype(jnp.float32), emb_ref[...],
                 preferred_element_type=jnp.float32)
    y_ref[...] = dinv * jnp.dot(h0, w_ref[...],
                                preferred_element_type=jnp.float32)
    dinv_ref[...] = dinv


def _ml0(x2, degT, emb, w0):
    return pl.pallas_call(
        _ml0_body,
        grid=(N // BN_BLK,),
        in_specs=[
            pl.BlockSpec((BN_BLK, 1), lambda i: (i, 0)),
            pl.BlockSpec((BN_BLK, NC), lambda i: (i, 0)),
            pl.BlockSpec((8, D), lambda i: (0, 0)),
            pl.BlockSpec((D, D), lambda i: (0, 0)),
        ],
        out_specs=[
            pl.BlockSpec((BN_BLK, D), lambda i: (i, 0)),
            pl.BlockSpec((BN_BLK, 1), lambda i: (i, 0)),
        ],
        out_shape=[
            jax.ShapeDtypeStruct((N, D), jnp.float32),
            jax.ShapeDtypeStruct((N, 1), jnp.float32),
        ],
    )(x2, degT, emb, w0)


def _ml_body(agg_ref, yp_ref, dinv_ref, w_ref, bb, rmr, rvr, gar, ber, y_ref):
    dinv = dinv_ref[...]
    t = dinv * (agg_ref[...] + yp_ref[...])
    sc = gar[...] * lax.rsqrt(rvr[...] + EPS)
    hh = (t + bb[...] - rmr[...]) * sc + ber[...]
    hh = jnp.maximum(hh, 0.0)
    y_ref[...] = dinv * jnp.dot(hh, w_ref[...],
                                preferred_element_type=jnp.float32)


def _ml(agg, yp, dinv, wl, bb, rmr, rvr, gar, ber):
    return pl.pallas_call(
        _ml_body,
        grid=(N // BN_BLK,),
        in_specs=[
            pl.BlockSpec((BN_BLK, D), lambda i: (i, 0)),
            pl.BlockSpec((BN_BLK, D), lambda i: (i, 0)),
            pl.BlockSpec((BN_BLK, 1), lambda i: (i, 0)),
            pl.BlockSpec((D, D), lambda i: (0, 0)),
            pl.BlockSpec((1, D), lambda i: (0, 0)),
            pl.BlockSpec((1, D), lambda i: (0, 0)),
            pl.BlockSpec((1, D), lambda i: (0, 0)),
            pl.BlockSpec((1, D), lambda i: (0, 0)),
            pl.BlockSpec((1, D), lambda i: (0, 0)),
        ],
        out_specs=pl.BlockSpec((BN_BLK, D), lambda i: (i, 0)),
        out_shape=jax.ShapeDtypeStruct((N, D), jnp.float32),
    )(agg, yp, dinv, wl, bb, rmr, rvr, gar, ber)


def _fin_body(agg_ref, yp_ref, dinv_ref, bb, rmr, rvr, gar, ber, o_ref):
    dinv = dinv_ref[...]
    t = dinv * (agg_ref[...] + yp_ref[...])
    sc = gar[...] * lax.rsqrt(rvr[...] + EPS)
    o_ref[...] = (t + bb[...] - rmr[...]) * sc + ber[...]


def _fin(agg, yp, dinv, bb, rmr, rvr, gar, ber):
    return pl.pallas_call(
        _fin_body,
        grid=(N // BN_BLK,),
        in_specs=[
            pl.BlockSpec((BN_BLK, D), lambda i: (i, 0)),
            pl.BlockSpec((BN_BLK, D), lambda i: (i, 0)),
            pl.BlockSpec((BN_BLK, 1), lambda i: (i, 0)),
            pl.BlockSpec((1, D), lambda i: (0, 0)),
            pl.BlockSpec((1, D), lambda i: (0, 0)),
            pl.BlockSpec((1, D), lambda i: (0, 0)),
            pl.BlockSpec((1, D), lambda i: (0, 0)),
            pl.BlockSpec((1, D), lambda i: (0, 0)),
        ],
        out_specs=pl.BlockSpec((BN_BLK, D), lambda i: (i, 0)),
        out_shape=jax.ShapeDtypeStruct((N, D), jnp.float32),
    )(agg, yp, dinv, bb, rmr, rvr, gar, ber)


def kernel(x, edge_index, edge_attr, batch, node_emb, W, b, gamma, beta, rm, rv):
    rowf = edge_index[0]
    colf = edge_index[1]
    prow, psid, pew, cnts, degp = _partition(rowf, colf, edge_attr)
    prow2 = prow.reshape(NB, NW, NCH, CH)
    psid2 = psid.reshape(NB, NW, NCH, CH)
    pew2 = pew.reshape(NB, NW, NCH, CH)
    cntsf = cnts.reshape(NW * NB)
    degT = degp[:, :N].T
    x2 = x.reshape(N, 1).astype(jnp.int32)

    y, dinv = _ml0(x2, degT, node_emb, W[0])
    p = lambda a, l: a[l].reshape(1, D)
    for l in range(1, NLAYER):
        agg = _aggregate(y, prow2, psid2, pew2, cntsf)
        y = _ml(agg, y, dinv, W[l], p(b, l - 1), p(rm, l - 1), p(rv, l - 1),
                p(gamma, l - 1), p(beta, l - 1))
    agg = _aggregate(y, prow2, psid2, pew2, cntsf)
    return _fin(agg, y, dinv, p(b, NLAYER - 1), p(rm, NLAYER - 1),
                p(rv, NLAYER - 1), p(gamma, NLAYER - 1), p(beta, NLAYER - 1))


# speedup vs baseline: 2.6350x; 2.6350x over previous
import dataclasses

import jax
import jax.numpy as jnp
from jax import lax
from jax.experimental import pallas as pl
from jax.experimental.pallas import tpu as pltpu
from jax.experimental.pallas import tpu_sc as plsc

N = 10000
E = 160000
D = 256
NLAYER = 5
EPS = 1e-5

NC = 2
NS = 16
NW = NC * NS
NB = NW
ACC_R = 320
OUTR = NB * ACC_R
EPW = E // NW
CAPP = 5120
CH = 64
NCH = CAPP // CH
DP = 257
DEG_ROWS = N + 240
DPS = DEG_ROWS // NS
BN_BLK = 1000

_mesh = plsc.VectorSubcoreMesh(core_axis_name="c", subcore_axis_name="s")

_sc_params = pltpu.CompilerParams()
if "needs_layout_passes" in pltpu.CompilerParams.__dataclass_fields__:
    _sc_params = dataclasses.replace(_sc_params, needs_layout_passes=False)


def _part_body(rowf, colf, ewf, prow, psid, pew, cnts, degp,
               rowb, colb, ewb, lrB, lsB, lwB,
               cntb, stage, idxb, zb, deg_sh):
    c = lax.axis_index("c")
    s = lax.axis_index("s")
    w = c * NS + s
    io = lax.iota(jnp.int32, 16)
    zeros16 = jnp.zeros((16,), jnp.float32)
    izeros16 = jnp.zeros((16,), jnp.int32)
    dummy_s = io & 7
    dummy_g = N + (io & 7)

    @pl.loop(0, DPS // 16)
    def _(i):
        zb[pl.ds(i * 16, 16)] = zeros16

    pltpu.sync_copy(zb, deg_sh.at[pl.ds(s * DPS, DPS)])

    pltpu.sync_copy(rowf.at[pl.ds(w * EPW, EPW)], rowb.at[pl.ds(0, EPW)])
    pltpu.sync_copy(colf.at[pl.ds(w * EPW, EPW)], colb.at[pl.ds(0, EPW)])
    pltpu.sync_copy(ewf.at[pl.ds(w * EPW, EPW)], ewb.at[pl.ds(0, EPW)])

    @pl.loop(0, CAPP // 16)
    def _(i):
        sl = pl.ds(i * 16, 16)
        lrB[sl] = izeros16
        lsB[sl] = dummy_s
        lwB[sl] = zeros16

    plsc.subcore_barrier()

    def deg_body(i, carry):
        sl = pl.ds(i * 16, 16)
        c16 = colb[sl]
        e16 = ewb[sl]
        valid = (i * 16 + io) < EPW
        idxb[0, pl.ds(0, 16)] = jnp.where(valid, c16, dummy_g)
        stage[...] = jnp.where(valid, e16, 0.0)
        pltpu.sync_copy(stage, deg_sh.at[idxb.at[0]], add=True)
        return carry

    lax.fori_loop(0, (EPW + 15) // 16, deg_body, jnp.int32(0))

    cv_lo = izeros16
    cv_hi = izeros16
    for t in range(NB):
        def body(i, off):
            sl = pl.ds(i * 16, 16)
            r16 = rowb[sl]
            c16 = colb[sl]
            e16 = ewb[sl]
            valid = (i * 16 + io) < EPW
            t16 = lax.div(c16, jnp.int32(ACC_R))
            m = valid & (t16 == t)
            plsc.store_compressed(lrB.at[pl.ds(off, 16)], r16, mask=m)
            plsc.store_compressed(lsB.at[pl.ds(off, 16)],
                                  c16 - (t * ACC_R), mask=m)
            plsc.store_compressed(lwB.at[pl.ds(off, 16)], e16, mask=m)
            return off + jnp.sum(m.astype(jnp.int32))

        off_t = lax.fori_loop(0, (EPW + 15) // 16, body, jnp.int32(0))

        pltpu.sync_copy(lrB, prow.at[t, w])
        pltpu.sync_copy(lsB, psid.at[t, w])
        pltpu.sync_copy(lwB, pew.at[t, w])

        def clr(i, carry):
            sl = pl.ds(i * 16, 16)
            lrB[sl] = izeros16
            lsB[sl] = dummy_s
            lwB[sl] = zeros16
            return carry

        lax.fori_loop(0, (off_t + 15) // 16, clr, jnp.int32(0))

        if t < NS:
            cv_lo = cv_lo + jnp.where(io == t, off_t, 0)
        else:
            cv_hi = cv_hi + jnp.where(io == (t - NS), off_t, 0)

    cntb[pl.ds(0, 16)] = cv_lo
    cntb[pl.ds(16, 16)] = cv_hi
    pltpu.sync_copy(cntb, cnts.at[w])

    plsc.subcore_barrier()
    pltpu.sync_copy(deg_sh.at[pl.ds(s * DPS, DPS)],
                    degp.at[c, pl.ds(s * DPS, DPS)])


def _partition(rowf, colf, ewf):
    k = pl.kernel(
        _part_body,
        out_type=[
            jax.ShapeDtypeStruct((NB, NW, CAPP), jnp.int32),
            jax.ShapeDtypeStruct((NB, NW, CAPP), jnp.int32),
            jax.ShapeDtypeStruct((NB, NW, CAPP), jnp.float32),
            jax.ShapeDtypeStruct((NW, NB), jnp.int32),
            jax.ShapeDtypeStruct((NC, DEG_ROWS), jnp.float32),
        ],
        mesh=_mesh,
        scratch_types=[
            pltpu.VMEM((EPW + 8,), jnp.int32),
            pltpu.VMEM((EPW + 8,), jnp.int32),
            pltpu.VMEM((EPW + 8,), jnp.float32),
            pltpu.VMEM((CAPP,), jnp.int32),
            pltpu.VMEM((CAPP,), jnp.int32),
            pltpu.VMEM((CAPP,), jnp.float32),
            pltpu.VMEM((32,), jnp.int32),
            pltpu.VMEM((16,), jnp.float32),
            pltpu.VMEM((1, 16), jnp.int32),
            pltpu.VMEM((DPS,), jnp.float32),
            pltpu.VMEM_SHARED((DEG_ROWS,), jnp.float32),
        ],
        compiler_params=_sc_params,
    )
    return k(rowf, colf, ewf)


def _agg_body(y, prow, psid, pew, cnts, out,
              gb, ridxb, sidxb, ewxb, cntv, acc):
    c = lax.axis_index("c")
    s = lax.axis_index("s")
    t = c * NS + s
    io = lax.iota(jnp.int32, 16)
    zeros16 = jnp.zeros((16,), jnp.float32)
    z16 = jnp.zeros((16,), jnp.int32)

    @pl.loop(0, ACC_R * DP // 16)
    def _(r):
        acc[pl.ds(r * 16, 16)] = zeros16

    pltpu.sync_copy(cnts, cntv)

    @pl.loop(0, NW)
    def _(w):
        cvec = plsc.load_gather(cntv, [jnp.full((16,), NB, jnp.int32) * w + t])
        cnt = jnp.max(cvec)
        nch = (cnt + (CH - 1)) // CH

        def chunk(j, carry):
            pltpu.sync_copy(prow.at[t, w, j], ridxb.at[0])
            pltpu.sync_copy(psid.at[t, w, j], sidxb.at[0])
            pltpu.sync_copy(pew.at[t, w, j], ewxb.at[0])
            pltpu.sync_copy(y.at[ridxb.at[0]], gb)
            ne = jnp.minimum(cnt - j * CH, CH)

            def edge(e, carry2):
                e16 = jnp.full((16,), e, jnp.int32)
                sid_spl = plsc.load_gather(sidxb, [z16, e16])
                ew_spl = plsc.load_gather(ewxb, [z16, e16])
                base16 = sid_spl * DP + io
                for dd in range(D // 16):
                    val = gb[e, pl.ds(dd * 16, 16)] * ew_spl
                    plsc.addupdate_scatter(acc, [base16 + dd * 16], val)
                return carry2

            lax.fori_loop(0, ne, edge, jnp.int32(0))
            return carry

        lax.fori_loop(0, nch, chunk, jnp.int32(0))

    @pl.loop(0, ACC_R // CH)
    def _(q):
        @pl.loop(0, CH)
        def _(r):
            for dd in range(D // 16):
                gb[r, pl.ds(dd * 16, 16)] = acc[
                    pl.ds((q * CH + r) * DP + dd * 16, 16)]
        pltpu.sync_copy(gb, out.at[pl.ds(t * ACC_R + q * CH, CH)])


def _aggregate(y, prow2, psid2, pew2, cnts):
    k = pl.kernel(
        _agg_body,
        out_type=jax.ShapeDtypeStruct((OUTR, D), jnp.float32),
        mesh=_mesh,
        scratch_types=[
            pltpu.VMEM((CH, D), jnp.float32),
            pltpu.VMEM((1, CH), jnp.int32),
            pltpu.VMEM((1, CH), jnp.int32),
            pltpu.VMEM((1, CH), jnp.float32),
            pltpu.VMEM((NW * NB,), jnp.int32),
            pltpu.VMEM((ACC_R * DP,), jnp.float32),
        ],
        compiler_params=_sc_params,
    )
    return k(y, prow2, psid2, pew2, cnts)


def _ml0_body(x_ref, degt_ref, emb_ref, w_ref, y_ref, dinv_ref):
    deg = jnp.sum(degt_ref[...], axis=1, keepdims=True) + 1.0
    dinv = jnp.where(deg > 0,
                     lax.rsqrt(jnp.maximum(deg, 1e-12)),
                     0.0)
    oh = (x_ref[...] == lax.broadcasted_iota(jnp.int32, (BN_BLK, 8), 1))
    h0 = jnp.dot(oh.astype(jnp.float32), emb_ref[...],
                 preferred_element_type=jnp.float32)
    y_ref[...] = dinv * jnp.dot(h0, w_ref[...],
                                preferred_element_type=jnp.float32)
    dinv_ref[...] = dinv


def _ml0(x2, degT, emb, w0):
    return pl.pallas_call(
        _ml0_body,
        grid=(N // BN_BLK,),
        in_specs=[
            pl.BlockSpec((BN_BLK, 1), lambda i: (i, 0)),
            pl.BlockSpec((BN_BLK, NC), lambda i: (i, 0)),
            pl.BlockSpec((8, D), lambda i: (0, 0)),
            pl.BlockSpec((D, D), lambda i: (0, 0)),
        ],
        out_specs=[
            pl.BlockSpec((BN_BLK, D), lambda i: (i, 0)),
            pl.BlockSpec((BN_BLK, 1), lambda i: (i, 0)),
        ],
        out_shape=[
            jax.ShapeDtypeStruct((N, D), jnp.float32),
            jax.ShapeDtypeStruct((N, 1), jnp.float32),
        ],
    )(x2, degT, emb, w0)


def _ml_body(agg_ref, yp_ref, dinv_ref, w_ref, bb, rmr, rvr, gar, ber, y_ref):
    dinv = dinv_ref[...]
    t = dinv * (agg_ref[...] + yp_ref[...])
    sc = gar[...] * lax.rsqrt(rvr[...] + EPS)
    hh = (t + bb[...] - rmr[...]) * sc + ber[...]
    hh = jnp.maximum(hh, 0.0)
    y_ref[...] = dinv * jnp.dot(hh, w_ref[...],
                                preferred_element_type=jnp.float32)


def _ml(agg, yp, dinv, wl, bb, rmr, rvr, gar, ber):
    return pl.pallas_call(
        _ml_body,
        grid=(N // BN_BLK,),
        in_specs=[
            pl.BlockSpec((BN_BLK, D), lambda i: (i, 0)),
            pl.BlockSpec((BN_BLK, D), lambda i: (i, 0)),
            pl.BlockSpec((BN_BLK, 1), lambda i: (i, 0)),
            pl.BlockSpec((D, D), lambda i: (0, 0)),
            pl.BlockSpec((1, D), lambda i: (0, 0)),
            pl.BlockSpec((1, D), lambda i: (0, 0)),
            pl.BlockSpec((1, D), lambda i: (0, 0)),
            pl.BlockSpec((1, D), lambda i: (0, 0)),
            pl.BlockSpec((1, D), lambda i: (0, 0)),
        ],
        out_specs=pl.BlockSpec((BN_BLK, D), lambda i: (i, 0)),
        out_shape=jax.ShapeDtypeStruct((N, D), jnp.float32),
    )(agg, yp, dinv, wl, bb, rmr, rvr, gar, ber)


def _fin_body(agg_ref, yp_ref, dinv_ref, bb, rmr, rvr, gar, ber, o_ref):
    dinv = dinv_ref[...]
    t = dinv * (agg_ref[...] + yp_ref[...])
    sc = gar[...] * lax.rsqrt(rvr[...] + EPS)
    o_ref[...] = (t + bb[...] - rmr[...]) * sc + ber[...]


def _fin(agg, yp, dinv, bb, rmr, rvr, gar, ber):
    return pl.pallas_call(
        _fin_body,
        grid=(N // BN_BLK,),
        in_specs=[
            pl.BlockSpec((BN_BLK, D), lambda i: (i, 0)),
            pl.BlockSpec((BN_BLK, D), lambda i: (i, 0)),
            pl.BlockSpec((BN_BLK, 1), lambda i: (i, 0)),
            pl.BlockSpec((1, D), lambda i: (0, 0)),
            pl.BlockSpec((1, D), lambda i: (0, 0)),
            pl.BlockSpec((1, D), lambda i: (0, 0)),
            pl.BlockSpec((1, D), lambda i: (0, 0)),
            pl.BlockSpec((1, D), lambda i: (0, 0)),
        ],
        out_specs=pl.BlockSpec((BN_BLK, D), lambda i: (i, 0)),
        out_shape=jax.ShapeDtypeStruct((N, D), jnp.float32),
    )(agg, yp, dinv, bb, rmr, rvr, gar, ber)


def kernel(x, edge_index, edge_attr, batch, node_emb, W, b, gamma, beta, rm, rv):
    rowf = edge_index[0]
    colf = edge_index[1]
    prow, psid, pew, cnts, degp = _partition(rowf, colf, edge_attr)
    prow2 = prow.reshape(NB, NW, NCH, CH)
    psid2 = psid.reshape(NB, NW, NCH, CH)
    pew2 = pew.reshape(NB, NW, NCH, CH)
    cntsf = cnts.reshape(NW * NB)
    degT = degp[:, :N].T
    x2 = x.reshape(N, 1).astype(jnp.int32)

    y, dinv = _ml0(x2, degT, node_emb, W[0])
    p = lambda a, l: a[l].reshape(1, D)
    for l in range(1, NLAYER):
        agg = _aggregate(y, prow2, psid2, pew2, cntsf)
        y = _ml(agg, y, dinv, W[l], p(b, l - 1), p(rm, l - 1), p(rv, l - 1),
                p(gamma, l - 1), p(beta, l - 1))
    agg = _aggregate(y, prow2, psid2, pew2, cntsf)
    return _fin(agg, y, dinv, p(b, NLAYER - 1), p(rm, NLAYER - 1),
                p(rv, NLAYER - 1), p(gamma, NLAYER - 1), p(beta, NLAYER - 1))
